# Initial kernel scaffold; baseline (speedup 1.0000x reference)
import jax
import jax.numpy as jnp
from jax.experimental import pallas as pl


def kernel(input, edge_index, W, a):
    raise NotImplementedError("write your pallas kernel here")


# speedup vs baseline: 2.7429x; 2.7429x over previous
import jax
import jax.numpy as jnp
from jax import lax
from jax.experimental import pallas as pl
from jax.experimental.pallas import tpu as pltpu
from jax.experimental.pallas import tpu_sc as plsc

N = 10000
E = 320000
D = 128
ALPHA = 0.2
SCALE = 10.0

NC = 2
NS = 16
L = 16
EPT = E // NS
CH = 80
NCHUNK = EPT // CH
SB = 50
NSC = NCHUNK // SB
NP = 10240
HN = NP // NC
RPT = HN // NS
MM_B = 1000


def _mm_body(x_ref, wt_ref, a12_ref, h_ref, s_ref):
    h = jnp.dot(x_ref[...], wt_ref[...], preferred_element_type=jnp.float32)
    h_ref[...] = h
    s_ref[...] = jnp.dot(h, a12_ref[...], preferred_element_type=jnp.float32)


def _mm(x, wt, a12):
    return pl.pallas_call(
        _mm_body,
        grid=(N // MM_B,),
        in_specs=[
            pl.BlockSpec((MM_B, D), lambda i: (i, 0)),
            pl.BlockSpec((D, D), lambda i: (0, 0)),
            pl.BlockSpec((D, 8), lambda i: (0, 0)),
        ],
        out_specs=[
            pl.BlockSpec((MM_B, D), lambda i: (i, 0)),
            pl.BlockSpec((MM_B, 8), lambda i: (i, 0)),
        ],
        out_shape=[
            jax.ShapeDtypeStruct((N, D), jnp.float32),
            jax.ShapeDtypeStruct((N, 8), jnp.float32),
        ],
    )(x, wt, a12)


def _sc_edge_body(h_hbm, src_hbm, dst_hbm, s1_hbm, s2_hbm,
                  out_hbm, rs_hbm,
                  s1_v, s2_v, srcv, dstv, w_v, src_m, dst_m,
                  rows_v, rows2_v, rs_local, probe_v,
                  acc_sh, sem):
    c = lax.axis_index("c")
    s = lax.axis_index("s")
    lane = lax.iota(jnp.int32, L)
    zero16 = jnp.zeros((L,), jnp.float32)
    lo = c * HN

    pltpu.sync_copy(s1_hbm, s1_v)
    pltpu.sync_copy(s2_hbm, s2_v)

    def _z16(i, carry):
        off = pl.multiple_of(i * L, L)
        rs_local[pl.ds(off, L)] = zero16
        return carry

    lax.fori_loop(0, HN // L, _z16, 0)

    def _zrow(i, carry):
        for u in range(D // L):
            rows2_v[i, pl.ds(u * L, L)] = zero16
        return carry

    lax.fori_loop(0, CH, _zrow, 0)
    for k in range(RPT // CH):
        pltpu.sync_copy(rows2_v, acc_sh.at[pl.ds(s * RPT + k * CH, CH)])
    plsc.subcore_barrier()

    def _chunk(j, carry):
        for g in range(CH // L):
            sl = pl.ds(g * L, L)
            sv = srcv[j, sl]
            dv = dstv[j, sl]
            inr = (sv >= lo) & (sv < lo + HN)
            shalf = sv - lo
            src_m[sl] = jnp.where(inr, shalf, -1)
            dst_m[sl] = jnp.where(inr, dv, -1)
            lg = plsc.load_gather(s1_v, [sv]) + plsc.load_gather(s2_v, [dv])
            w = jnp.exp(-jnp.where(lg > 0, lg, ALPHA * lg))
            w_v[sl] = w
            si = jnp.where(inr, shalf, 0)

            def _rs_body(m):
                plsc.store_scatter(probe_v, [si], lane, mask=m)
                got = plsc.load_gather(probe_v, [si])
                win = m & (got == lane)
                plsc.addupdate_scatter(rs_local, [si], w, mask=win)
                return m & jnp.logical_not(win)

            lax.while_loop(
                lambda m: jnp.sum(m.astype(jnp.int32)) > 0,
                _rs_body,
                inr,
            )

        pltpu.async_copy(
            h_hbm.at[plsc.Indices(dst_m, ignored_value=-1)], rows_v, sem
        ).wait()

        def _row(r, carry2):
            wspl = plsc.load_gather(w_v, [jnp.full((L,), r, jnp.int32)])
            for u in range(D // L):
                rows2_v[r, pl.ds(u * L, L)] = rows_v[r, pl.ds(u * L, L)] * wspl
            return carry2

        lax.fori_loop(0, CH, _row, 0)
        pltpu.sync_copy(
            rows2_v, acc_sh.at[plsc.Indices(src_m, ignored_value=-1)],
            add=True)
        return carry

    for k in range(NSC):
        pltpu.sync_copy(src_hbm.at[s, k], srcv)
        pltpu.sync_copy(dst_hbm.at[s, k], dstv)
        lax.fori_loop(0, SB, _chunk, 0)

    pltpu.sync_copy(rs_local, rs_hbm.at[s, 0, pl.ds(c * HN, HN)])
    plsc.subcore_barrier()
    pltpu.sync_copy(acc_sh.at[pl.ds(s * RPT, RPT)],
                    out_hbm.at[pl.ds(c * HN + s * RPT, RPT)])


def _sc_edge(h, src3, dst3, s1, s2):
    mesh = plsc.VectorSubcoreMesh(core_axis_name="c", subcore_axis_name="s",
                                  num_cores=NC, num_subcores=NS)
    fn = pl.kernel(
        _sc_edge_body,
        out_type=[
            jax.ShapeDtypeStruct((NP, D), jnp.float32),
            jax.ShapeDtypeStruct((NS, 1, NP), jnp.float32),
        ],
        mesh=mesh,
        scratch_types=[
            pltpu.VMEM((N,), jnp.float32),
            pltpu.VMEM((N,), jnp.float32),
            pltpu.VMEM((SB, CH), jnp.int32),
            pltpu.VMEM((SB, CH), jnp.int32),
            pltpu.VMEM((CH,), jnp.float32),
            pltpu.VMEM((CH,), jnp.int32),
            pltpu.VMEM((CH,), jnp.int32),
            pltpu.VMEM((CH, D), jnp.float32),
            pltpu.VMEM((CH, D), jnp.float32),
            pltpu.VMEM((HN,), jnp.float32),
            pltpu.VMEM((HN,), jnp.int32),
            pltpu.VMEM_SHARED((HN, D), jnp.float32),
            pltpu.SemaphoreType.DMA,
        ],
        compiler_params=pltpu.CompilerParams(needs_layout_passes=False),
    )
    return fn(h, src3, dst3, s1, s2)


def _fin_body(acc_ref, rs_ref, o_ref):
    hp = acc_ref[...]
    rs = jnp.sum(rs_ref[...], axis=0)[:, None]
    hp = hp / (rs + 1e-16)
    out = jnp.where(hp > 0, hp, jnp.exp(jnp.minimum(hp, 0.0)) - 1.0)
    u = out / SCALE
    nrm = jnp.maximum(
        jnp.sqrt(jnp.sum(u * u, axis=-1, keepdims=True)), 1e-15)
    p = jnp.tanh(nrm) * u / nrm
    pn = jnp.maximum(
        jnp.sqrt(jnp.sum(p * p, axis=-1, keepdims=True)), 1e-15)
    maxnorm = 1.0 - 1e-5
    o_ref[...] = jnp.where(pn > maxnorm, p / pn * maxnorm, p)


FIN_B = 1024


def _fin(acc, rs):
    return pl.pallas_call(
        _fin_body,
        grid=(NP // FIN_B,),
        in_specs=[
            pl.BlockSpec((FIN_B, D), lambda i: (i, 0)),
            pl.BlockSpec((NS, FIN_B), lambda i: (0, i)),
        ],
        out_specs=pl.BlockSpec((FIN_B, D), lambda i: (i, 0)),
        out_shape=jax.ShapeDtypeStruct((NP, D), jnp.float32),
    )(acc, rs)


def kernel(input, edge_index, W, a):
    x = input.astype(jnp.float32)
    wt = W.T
    a12 = jnp.pad(a.reshape(2, D).T, ((0, 0), (0, 6)))
    h, sprj = _mm(x, wt, a12)
    s1 = sprj[:, 0]
    s2 = sprj[:, 1]
    src3 = edge_index[0].reshape(NS, NSC, SB, CH)
    dst3 = edge_index[1].reshape(NS, NSC, SB, CH)
    acc, rs = _sc_edge(h, src3, dst3, s1, s2)
    return _fin(acc, rs.reshape(NS, NP))[:N]


# speedup vs baseline: 7.4853x; 2.7290x over previous
import jax
import jax.numpy as jnp
from jax import lax
from jax.experimental import pallas as pl
from jax.experimental.pallas import tpu as pltpu
from jax.experimental.pallas import tpu_sc as plsc

N = 10000
E = 320000
D = 128
ALPHA = 0.2
SCALE = 10.0

NC = 2
NS = 16
L = 16
EPT = E // NS
CH = 80
NCHUNK = EPT // CH
SB = 50
NSC = NCHUNK // SB
NP = 10240
HN = NP // NC
RPT = HN // NS
MM_B = 1000


def _mm_body(x_ref, wt_ref, a12_ref, h_ref, s_ref):
    h = jnp.dot(x_ref[...], wt_ref[...], preferred_element_type=jnp.float32)
    h_ref[...] = h
    s_ref[...] = jnp.dot(h, a12_ref[...], preferred_element_type=jnp.float32)


def _mm(x, wt, a12):
    return pl.pallas_call(
        _mm_body,
        grid=(N // MM_B,),
        in_specs=[
            pl.BlockSpec((MM_B, D), lambda i: (i, 0)),
            pl.BlockSpec((D, D), lambda i: (0, 0)),
            pl.BlockSpec((D, 8), lambda i: (0, 0)),
        ],
        out_specs=[
            pl.BlockSpec((MM_B, D), lambda i: (i, 0)),
            pl.BlockSpec((MM_B, 8), lambda i: (i, 0)),
        ],
        out_shape=[
            jax.ShapeDtypeStruct((N, D), jnp.float32),
            jax.ShapeDtypeStruct((N, 8), jnp.float32),
        ],
    )(x, wt, a12)


def _sc_edge_body(h_hbm, src_hbm, dst_hbm, s1_hbm, s2_hbm,
                  out_hbm, rs_hbm,
                  s1_v, s2_v, srcv, dstv,
                  w_a, w_b, srcm_a, srcm_b, dstm_a, dstm_b,
                  rows_a, rows_b, rs_local, probe_v,
                  acc_sh, sg_a, sg_b, ss_a, ss_b):
    c = lax.axis_index("c")
    s = lax.axis_index("s")
    lane = lax.iota(jnp.int32, L)
    zero16 = jnp.zeros((L,), jnp.float32)
    lo = c * HN

    pltpu.sync_copy(s1_hbm, s1_v)
    pltpu.sync_copy(s2_hbm, s2_v)

    def _z16(i, carry):
        off = pl.multiple_of(i * L, L)
        rs_local[pl.ds(off, L)] = zero16
        return carry

    lax.fori_loop(0, HN // L, _z16, 0)

    def _zrow(i, carry):
        for u in range(D // L):
            rows_a[i, pl.ds(u * L, L)] = zero16
        return carry

    lax.fori_loop(0, CH, _zrow, 0)
    for k in range(RPT // CH):
        pltpu.sync_copy(rows_a, acc_sh.at[pl.ds(s * RPT + k * CH, CH)])
    plsc.subcore_barrier()

    def _masks_w(j, w_v, src_m, dst_m):
        for g in range(CH // L):
            sl = pl.ds(g * L, L)
            sv = srcv[j, sl]
            dv = dstv[j, sl]
            inr = (sv >= lo) & (sv < lo + HN)
            shalf = sv - lo
            src_m[sl] = jnp.where(inr, shalf, -1)
            dst_m[sl] = jnp.where(inr, dv, -1)
            lg = plsc.load_gather(s1_v, [sv]) + plsc.load_gather(s2_v, [dv])
            w = jnp.exp(-jnp.where(lg > 0, lg, ALPHA * lg))
            w_v[sl] = w
            si = jnp.where(inr, shalf, 0)

            def _rs_body(m):
                plsc.store_scatter(probe_v, [si], lane, mask=m)
                got = plsc.load_gather(probe_v, [si])
                win = m & (got == lane)
                plsc.addupdate_scatter(rs_local, [si], w, mask=win)
                return m & jnp.logical_not(win)

            lax.while_loop(
                lambda m: jnp.sum(m.astype(jnp.int32)) > 0,
                _rs_body,
                inr,
            )

    def _scale(w_v, rows_v):
        def _row(r, carry2):
            wspl = plsc.load_gather(w_v, [jnp.full((L,), r, jnp.int32)])
            for u in range(D // L):
                rows_v[r, pl.ds(u * L, L)] = rows_v[r, pl.ds(u * L, L)] * wspl
            return carry2

        lax.fori_loop(0, CH, _row, 0)

    def _g_start(dst_m, rows_v, sem):
        pltpu.async_copy(
            h_hbm.at[plsc.Indices(dst_m, ignored_value=-1)], rows_v, sem)

    def _g_wait(dst_m, rows_v, sem):
        pltpu.make_async_copy(
            h_hbm.at[plsc.Indices(dst_m, ignored_value=-1)], rows_v, sem
        ).wait()

    def _s_start(src_m, rows_v, sem):
        pltpu.async_copy(
            rows_v, acc_sh.at[plsc.Indices(src_m, ignored_value=-1)], sem,
            add=True)

    def _s_wait(src_m, rows_v, sem):
        pltpu.make_async_copy(
            rows_v, acc_sh.at[plsc.Indices(src_m, ignored_value=-1)], sem
        ).wait()

    NPAIR = SB // 2
    for k in range(NSC):
        pltpu.sync_copy(src_hbm.at[s, k], srcv)
        pltpu.sync_copy(dst_hbm.at[s, k], dstv)

        _masks_w(0, w_a, srcm_a, dstm_a)
        _g_start(dstm_a, rows_a, sg_a)
        _masks_w(1, w_b, srcm_b, dstm_b)
        _g_start(dstm_b, rows_b, sg_b)

        def _pair(i, carry):
            _g_wait(dstm_a, rows_a, sg_a)
            _scale(w_a, rows_a)
            _s_start(srcm_a, rows_a, ss_a)
            _g_wait(dstm_b, rows_b, sg_b)
            _scale(w_b, rows_b)
            _s_start(srcm_b, rows_b, ss_b)

            @pl.when(i < NPAIR - 1)
            def _prep_next():
                _s_wait(srcm_a, rows_a, ss_a)
                _masks_w(2 * i + 2, w_a, srcm_a, dstm_a)
                _g_start(dstm_a, rows_a, sg_a)
                _s_wait(srcm_b, rows_b, ss_b)
                _masks_w(2 * i + 3, w_b, srcm_b, dstm_b)
                _g_start(dstm_b, rows_b, sg_b)

            return carry

        lax.fori_loop(0, NPAIR, _pair, 0)
        _s_wait(srcm_a, rows_a, ss_a)
        _s_wait(srcm_b, rows_b, ss_b)

    pltpu.sync_copy(rs_local, rs_hbm.at[s, 0, pl.ds(c * HN, HN)])
    plsc.subcore_barrier()
    pltpu.sync_copy(acc_sh.at[pl.ds(s * RPT, RPT)],
                    out_hbm.at[pl.ds(c * HN + s * RPT, RPT)])


def _sc_edge(h, src3, dst3, s1, s2):
    mesh = plsc.VectorSubcoreMesh(core_axis_name="c", subcore_axis_name="s",
                                  num_cores=NC, num_subcores=NS)
    fn = pl.kernel(
        _sc_edge_body,
        out_type=[
            jax.ShapeDtypeStruct((NP, D), jnp.float32),
            jax.ShapeDtypeStruct((NS, 1, NP), jnp.float32),
        ],
        mesh=mesh,
        scratch_types=[
            pltpu.VMEM((N,), jnp.float32),
            pltpu.VMEM((N,), jnp.float32),
            pltpu.VMEM((SB, CH), jnp.int32),
            pltpu.VMEM((SB, CH), jnp.int32),
            pltpu.VMEM((CH,), jnp.float32),
            pltpu.VMEM((CH,), jnp.float32),
            pltpu.VMEM((CH,), jnp.int32),
            pltpu.VMEM((CH,), jnp.int32),
            pltpu.VMEM((CH,), jnp.int32),
            pltpu.VMEM((CH,), jnp.int32),
            pltpu.VMEM((CH, D), jnp.float32),
            pltpu.VMEM((CH, D), jnp.float32),
            pltpu.VMEM((HN,), jnp.float32),
            pltpu.VMEM((HN,), jnp.int32),
            pltpu.VMEM_SHARED((HN, D), jnp.float32),
            pltpu.SemaphoreType.DMA,
            pltpu.SemaphoreType.DMA,
            pltpu.SemaphoreType.DMA,
            pltpu.SemaphoreType.DMA,
        ],
        compiler_params=pltpu.CompilerParams(needs_layout_passes=False),
    )
    return fn(h, src3, dst3, s1, s2)


def _fin_body(acc_ref, rs_ref, o_ref):
    hp = acc_ref[...]
    rs = jnp.sum(rs_ref[...], axis=0)[:, None]
    hp = hp / (rs + 1e-16)
    out = jnp.where(hp > 0, hp, jnp.exp(jnp.minimum(hp, 0.0)) - 1.0)
    u = out / SCALE
    nrm = jnp.maximum(
        jnp.sqrt(jnp.sum(u * u, axis=-1, keepdims=True)), 1e-15)
    p = jnp.tanh(nrm) * u / nrm
    pn = jnp.maximum(
        jnp.sqrt(jnp.sum(p * p, axis=-1, keepdims=True)), 1e-15)
    maxnorm = 1.0 - 1e-5
    o_ref[...] = jnp.where(pn > maxnorm, p / pn * maxnorm, p)


FIN_B = 1024


def _fin(acc, rs):
    return pl.pallas_call(
        _fin_body,
        grid=(NP // FIN_B,),
        in_specs=[
            pl.BlockSpec((FIN_B, D), lambda i: (i, 0)),
            pl.BlockSpec((NS, FIN_B), lambda i: (0, i)),
        ],
        out_specs=pl.BlockSpec((FIN_B, D), lambda i: (i, 0)),
        out_shape=jax.ShapeDtypeStruct((NP, D), jnp.float32),
    )(acc, rs)


def kernel(input, edge_index, W, a):
    x = input.astype(jnp.float32)
    wt = W.T
    a12 = jnp.pad(a.reshape(2, D).T, ((0, 0), (0, 6)))
    h, sprj = _mm(x, wt, a12)
    s1 = sprj[:, 0]
    s2 = sprj[:, 1]
    src3 = edge_index[0].reshape(NS, NSC, SB, CH)
    dst3 = edge_index[1].reshape(NS, NSC, SB, CH)
    acc, rs = _sc_edge(h, src3, dst3, s1, s2)
    return _fin(acc, rs.reshape(NS, NP))[:N]


# speedup vs baseline: 9.3789x; 1.2530x over previous
import jax
import jax.numpy as jnp
from jax import lax
from jax.experimental import pallas as pl
from jax.experimental.pallas import tpu as pltpu
from jax.experimental.pallas import tpu_sc as plsc

N = 10000
E = 320000
D = 128
ALPHA = 0.2
SCALE = 10.0

NC = 2
NS = 16
L = 16
EPT = E // NS
CH = 80
NCHUNK = EPT // CH
SB = 50
NSC = NCHUNK // SB
NP = 10240
HN = NP // NC
RPT = HN // NS
MM_B = 1000


def _mm_body(x_ref, wt_ref, a12_ref, h_ref, s_ref):
    h = jnp.dot(x_ref[...], wt_ref[...], preferred_element_type=jnp.float32)
    h_ref[...] = h
    s_ref[...] = jnp.dot(h, a12_ref[...], preferred_element_type=jnp.float32)


def _mm(x, wt, a12):
    return pl.pallas_call(
        _mm_body,
        grid=(N // MM_B,),
        in_specs=[
            pl.BlockSpec((MM_B, D), lambda i: (i, 0)),
            pl.BlockSpec((D, D), lambda i: (0, 0)),
            pl.BlockSpec((D, 8), lambda i: (0, 0)),
        ],
        out_specs=[
            pl.BlockSpec((MM_B, D), lambda i: (i, 0)),
            pl.BlockSpec((MM_B, 8), lambda i: (i, 0)),
        ],
        out_shape=[
            jax.ShapeDtypeStruct((N, D), jnp.float32),
            jax.ShapeDtypeStruct((N, 8), jnp.float32),
        ],
    )(x, wt, a12)


def _sc_edge_body(h_hbm, src_hbm, dst_hbm, s1_hbm, s2_hbm,
                  out_hbm, rs_hbm,
                  s1_v, s2_v, srcv, dstv,
                  w_a, w_b, srcm_a, srcm_b, dstm_a, dstm_b,
                  rows_a, rows_b, rs_local, probe_v,
                  acc_sh, sg_a, sg_b, ss_a, ss_b):
    c = lax.axis_index("c")
    s = lax.axis_index("s")
    lane = lax.iota(jnp.int32, L)
    zero16 = jnp.zeros((L,), jnp.float32)
    lo = c * HN

    pltpu.sync_copy(s1_hbm, s1_v)
    pltpu.sync_copy(s2_hbm, s2_v)

    def _z16(i, carry):
        off = pl.multiple_of(i * L, L)
        rs_local[pl.ds(off, L)] = zero16
        return carry

    lax.fori_loop(0, HN // L, _z16, 0)

    def _zrow(i, carry):
        for u in range(D // L):
            rows_a[i, pl.ds(u * L, L)] = zero16
        return carry

    lax.fori_loop(0, CH, _zrow, 0)
    for k in range(RPT // CH):
        pltpu.sync_copy(rows_a, acc_sh.at[pl.ds(s * RPT + k * CH, CH)])
    plsc.subcore_barrier()

    def _masks_w(j, w_v, src_m, dst_m):
        for g in range(CH // L):
            sl = pl.ds(g * L, L)
            sv = srcv[j, sl]
            dv = dstv[j, sl]
            inr = (sv >= lo) & (sv < lo + HN)
            shalf = sv - lo
            src_m[sl] = jnp.where(inr, shalf, -1)
            dst_m[sl] = jnp.where(inr, dv, -1)
            lg = plsc.load_gather(s1_v, [sv]) + plsc.load_gather(s2_v, [dv])
            w = jnp.exp(-jnp.where(lg > 0, lg, ALPHA * lg))
            w_v[sl] = w
            si = jnp.where(inr, shalf, 0)

            def _rs_body(m):
                plsc.store_scatter(probe_v, [si], lane, mask=m)
                got = plsc.load_gather(probe_v, [si])
                win = m & (got == lane)
                plsc.addupdate_scatter(rs_local, [si], w, mask=win)
                return m & jnp.logical_not(win)

            lax.while_loop(
                lambda m: jnp.sum(m.astype(jnp.int32)) > 0,
                _rs_body,
                inr,
            )

    def _scale(w_v, rows_v):
        @plsc.parallel_loop(0, CH, unroll=4)
        def _row(r):
            wspl = plsc.load_gather(w_v, [jnp.full((L,), r, jnp.int32)])
            for u in range(D // L):
                rows_v[r, pl.ds(u * L, L)] = rows_v[r, pl.ds(u * L, L)] * wspl

    def _g_start(dst_m, rows_v, sem):
        pltpu.async_copy(
            h_hbm.at[plsc.Indices(dst_m, ignored_value=-1)], rows_v, sem)

    def _g_wait(dst_m, rows_v, sem):
        pltpu.make_async_copy(
            h_hbm.at[plsc.Indices(dst_m, ignored_value=-1)], rows_v, sem
        ).wait()

    def _s_start(src_m, rows_v, sem):
        pltpu.async_copy(
            rows_v, acc_sh.at[plsc.Indices(src_m, ignored_value=-1)], sem,
            add=True)

    def _s_wait(src_m, rows_v, sem):
        pltpu.make_async_copy(
            rows_v, acc_sh.at[plsc.Indices(src_m, ignored_value=-1)], sem
        ).wait()

    NPAIR = SB // 2
    for k in range(NSC):
        pltpu.sync_copy(src_hbm.at[s, k], srcv)
        pltpu.sync_copy(dst_hbm.at[s, k], dstv)

        _masks_w(0, w_a, srcm_a, dstm_a)
        _g_start(dstm_a, rows_a, sg_a)
        _masks_w(1, w_b, srcm_b, dstm_b)
        _g_start(dstm_b, rows_b, sg_b)

        def _pair(i, carry):
            _g_wait(dstm_a, rows_a, sg_a)
            _scale(w_a, rows_a)
            _s_start(srcm_a, rows_a, ss_a)
            _g_wait(dstm_b, rows_b, sg_b)
            _scale(w_b, rows_b)
            _s_start(srcm_b, rows_b, ss_b)

            @pl.when(i < NPAIR - 1)
            def _prep_next():
                _s_wait(srcm_a, rows_a, ss_a)
                _masks_w(2 * i + 2, w_a, srcm_a, dstm_a)
                _g_start(dstm_a, rows_a, sg_a)
                _s_wait(srcm_b, rows_b, ss_b)
                _masks_w(2 * i + 3, w_b, srcm_b, dstm_b)
                _g_start(dstm_b, rows_b, sg_b)

            return carry

        lax.fori_loop(0, NPAIR, _pair, 0)
        _s_wait(srcm_a, rows_a, ss_a)
        _s_wait(srcm_b, rows_b, ss_b)

    pltpu.sync_copy(rs_local, rs_hbm.at[s, 0, pl.ds(c * HN, HN)])
    plsc.subcore_barrier()
    pltpu.sync_copy(acc_sh.at[pl.ds(s * RPT, RPT)],
                    out_hbm.at[pl.ds(c * HN + s * RPT, RPT)])


def _sc_edge(h, src3, dst3, s1, s2):
    mesh = plsc.VectorSubcoreMesh(core_axis_name="c", subcore_axis_name="s",
                                  num_cores=NC, num_subcores=NS)
    fn = pl.kernel(
        _sc_edge_body,
        out_type=[
            jax.ShapeDtypeStruct((NP, D), jnp.float32),
            jax.ShapeDtypeStruct((NS, 1, NP), jnp.float32),
        ],
        mesh=mesh,
        scratch_types=[
            pltpu.VMEM((N,), jnp.float32),
            pltpu.VMEM((N,), jnp.float32),
            pltpu.VMEM((SB, CH), jnp.int32),
            pltpu.VMEM((SB, CH), jnp.int32),
            pltpu.VMEM((CH,), jnp.float32),
            pltpu.VMEM((CH,), jnp.float32),
            pltpu.VMEM((CH,), jnp.int32),
            pltpu.VMEM((CH,), jnp.int32),
            pltpu.VMEM((CH,), jnp.int32),
            pltpu.VMEM((CH,), jnp.int32),
            pltpu.VMEM((CH, D), jnp.float32),
            pltpu.VMEM((CH, D), jnp.float32),
            pltpu.VMEM((HN,), jnp.float32),
            pltpu.VMEM((HN,), jnp.int32),
            pltpu.VMEM_SHARED((HN, D), jnp.float32),
            pltpu.SemaphoreType.DMA,
            pltpu.SemaphoreType.DMA,
            pltpu.SemaphoreType.DMA,
            pltpu.SemaphoreType.DMA,
        ],
        compiler_params=pltpu.CompilerParams(needs_layout_passes=False),
    )
    return fn(h, src3, dst3, s1, s2)


def _fin_body(acc_ref, rs_ref, o_ref):
    hp = acc_ref[...]
    rs = jnp.sum(rs_ref[...], axis=0)[:, None]
    hp = hp / (rs + 1e-16)
    out = jnp.where(hp > 0, hp, jnp.exp(jnp.minimum(hp, 0.0)) - 1.0)
    u = out / SCALE
    nrm = jnp.maximum(
        jnp.sqrt(jnp.sum(u * u, axis=-1, keepdims=True)), 1e-15)
    p = jnp.tanh(nrm) * u / nrm
    pn = jnp.maximum(
        jnp.sqrt(jnp.sum(p * p, axis=-1, keepdims=True)), 1e-15)
    maxnorm = 1.0 - 1e-5
    o_ref[...] = jnp.where(pn > maxnorm, p / pn * maxnorm, p)


FIN_B = 1024


def _fin(acc, rs):
    return pl.pallas_call(
        _fin_body,
        grid=(NP // FIN_B,),
        in_specs=[
            pl.BlockSpec((FIN_B, D), lambda i: (i, 0)),
            pl.BlockSpec((NS, FIN_B), lambda i: (0, i)),
        ],
        out_specs=pl.BlockSpec((FIN_B, D), lambda i: (i, 0)),
        out_shape=jax.ShapeDtypeStruct((NP, D), jnp.float32),
    )(acc, rs)


def kernel(input, edge_index, W, a):
    x = input.astype(jnp.float32)
    wt = W.T
    a12 = jnp.pad(a.reshape(2, D).T, ((0, 0), (0, 6)))
    h, sprj = _mm(x, wt, a12)
    s1 = sprj[:, 0]
    s2 = sprj[:, 1]
    src3 = edge_index[0].reshape(NS, NSC, SB, CH)
    dst3 = edge_index[1].reshape(NS, NSC, SB, CH)
    acc, rs = _sc_edge(h, src3, dst3, s1, s2)
    return _fin(acc, rs.reshape(NS, NP))[:N]


# speedup vs baseline: 9.4297x; 1.0054x over previous
import jax
import jax.numpy as jnp
from jax import lax
from jax.experimental import pallas as pl
from jax.experimental.pallas import tpu as pltpu
from jax.experimental.pallas import tpu_sc as plsc

N = 10000
E = 320000
D = 128
ALPHA = 0.2
SCALE = 10.0

NC = 2
NS = 16
L = 16
EPT = E // NS
CH = 80
NCHUNK = EPT // CH
SB = 50
NSC = NCHUNK // SB
NP = 10240
HN = NP // NC
RPT = HN // NS
MM_B = 1000


def _mm_body(x_ref, wt_ref, a12_ref, h_ref, s_ref):
    h = jnp.dot(x_ref[...], wt_ref[...], preferred_element_type=jnp.float32)
    h_ref[...] = h
    s_ref[...] = jnp.dot(h, a12_ref[...], preferred_element_type=jnp.float32)


def _mm(x, wt, a12):
    return pl.pallas_call(
        _mm_body,
        grid=(N // MM_B,),
        in_specs=[
            pl.BlockSpec((MM_B, D), lambda i: (i, 0)),
            pl.BlockSpec((D, D), lambda i: (0, 0)),
            pl.BlockSpec((D, 8), lambda i: (0, 0)),
        ],
        out_specs=[
            pl.BlockSpec((MM_B, D), lambda i: (i, 0)),
            pl.BlockSpec((MM_B, 8), lambda i: (i, 0)),
        ],
        out_shape=[
            jax.ShapeDtypeStruct((N, D), jnp.float32),
            jax.ShapeDtypeStruct((N, 8), jnp.float32),
        ],
    )(x, wt, a12)


def _sc_edge_body(h_hbm, src_hbm, dst_hbm, s1_hbm, s2_hbm,
                  out_hbm, rs_hbm,
                  s1_v, s2_v, srcv, dstv,
                  w_a, w_b, srcm_a, srcm_b, dstm_a, dstm_b,
                  rows_a, rows_b, rs_local, probe_v,
                  acc_sh, sg_a, sg_b, ss_a, ss_b):
    c = lax.axis_index("c")
    s = lax.axis_index("s")
    lane = lax.iota(jnp.int32, L)
    zero16 = jnp.zeros((L,), jnp.float32)
    lo = c * HN

    pltpu.sync_copy(s1_hbm, s1_v)
    pltpu.sync_copy(s2_hbm, s2_v)

    def _z16(i, carry):
        off = pl.multiple_of(i * L, L)
        rs_local[pl.ds(off, L)] = zero16
        return carry

    lax.fori_loop(0, HN // L, _z16, 0)

    def _zrow(i, carry):
        for u in range(D // L):
            rows_a[i, pl.ds(u * L, L)] = zero16
        return carry

    lax.fori_loop(0, CH, _zrow, 0)
    for k in range(RPT // CH):
        pltpu.sync_copy(rows_a, acc_sh.at[pl.ds(s * RPT + k * CH, CH)])
    plsc.subcore_barrier()

    def _masks_w(j, w_v, src_m, dst_m):
        for g in range(CH // L):
            sl = pl.ds(g * L, L)
            sv = srcv[j, sl]
            dv = dstv[j, sl]
            inr = (sv >= lo) & (sv < lo + HN)
            shalf = sv - lo
            src_m[sl] = jnp.where(inr, shalf, -1)
            dst_m[sl] = jnp.where(inr, dv, -1)
            lg = plsc.load_gather(s1_v, [sv]) + plsc.load_gather(s2_v, [dv])
            w = jnp.exp(-jnp.where(lg > 0, lg, ALPHA * lg))
            w_v[sl] = w
            si = jnp.where(inr, shalf, 0)

            def _rs_round(m):
                plsc.store_scatter(probe_v, [si], lane, mask=m)
                got = plsc.load_gather(probe_v, [si])
                win = m & (got == lane)
                plsc.addupdate_scatter(rs_local, [si], w, mask=win)
                return m & jnp.logical_not(win)

            lax.while_loop(jnp.any, _rs_round, _rs_round(inr))

    def _scale(w_v, rows_v):
        @plsc.parallel_loop(0, CH, unroll=4)
        def _row(r):
            wspl = plsc.load_gather(w_v, [jnp.full((L,), r, jnp.int32)])
            for u in range(D // L):
                rows_v[r, pl.ds(u * L, L)] = rows_v[r, pl.ds(u * L, L)] * wspl

    def _g_start(dst_m, rows_v, sem):
        pltpu.async_copy(
            h_hbm.at[plsc.Indices(dst_m, ignored_value=-1)], rows_v, sem)

    def _g_wait(dst_m, rows_v, sem):
        pltpu.make_async_copy(
            h_hbm.at[plsc.Indices(dst_m, ignored_value=-1)], rows_v, sem
        ).wait()

    def _s_start(src_m, rows_v, sem):
        pltpu.async_copy(
            rows_v, acc_sh.at[plsc.Indices(src_m, ignored_value=-1)], sem,
            add=True)

    def _s_wait(src_m, rows_v, sem):
        pltpu.make_async_copy(
            rows_v, acc_sh.at[plsc.Indices(src_m, ignored_value=-1)], sem
        ).wait()

    NPAIR = SB // 2
    for k in range(NSC):
        pltpu.sync_copy(src_hbm.at[s, k], srcv)
        pltpu.sync_copy(dst_hbm.at[s, k], dstv)

        _masks_w(0, w_a, srcm_a, dstm_a)
        _g_start(dstm_a, rows_a, sg_a)
        _masks_w(1, w_b, srcm_b, dstm_b)
        _g_start(dstm_b, rows_b, sg_b)

        def _pair(i, carry):
            _g_wait(dstm_a, rows_a, sg_a)
            _scale(w_a, rows_a)
            _s_start(srcm_a, rows_a, ss_a)
            _g_wait(dstm_b, rows_b, sg_b)
            _scale(w_b, rows_b)
            _s_start(srcm_b, rows_b, ss_b)

            @pl.when(i < NPAIR - 1)
            def _prep_next():
                _s_wait(srcm_a, rows_a, ss_a)
                _masks_w(2 * i + 2, w_a, srcm_a, dstm_a)
                _g_start(dstm_a, rows_a, sg_a)
                _s_wait(srcm_b, rows_b, ss_b)
                _masks_w(2 * i + 3, w_b, srcm_b, dstm_b)
                _g_start(dstm_b, rows_b, sg_b)

            return carry

        lax.fori_loop(0, NPAIR, _pair, 0)
        _s_wait(srcm_a, rows_a, ss_a)
        _s_wait(srcm_b, rows_b, ss_b)

    pltpu.sync_copy(rs_local, rs_hbm.at[s, 0, pl.ds(c * HN, HN)])
    plsc.subcore_barrier()
    pltpu.sync_copy(acc_sh.at[pl.ds(s * RPT, RPT)],
                    out_hbm.at[pl.ds(c * HN + s * RPT, RPT)])


def _sc_edge(h, src3, dst3, s1, s2):
    mesh = plsc.VectorSubcoreMesh(core_axis_name="c", subcore_axis_name="s",
                                  num_cores=NC, num_subcores=NS)
    fn = pl.kernel(
        _sc_edge_body,
        out_type=[
            jax.ShapeDtypeStruct((NP, D), jnp.float32),
            jax.ShapeDtypeStruct((NS, 1, NP), jnp.float32),
        ],
        mesh=mesh,
        scratch_types=[
            pltpu.VMEM((N,), jnp.float32),
            pltpu.VMEM((N,), jnp.float32),
            pltpu.VMEM((SB, CH), jnp.int32),
            pltpu.VMEM((SB, CH), jnp.int32),
            pltpu.VMEM((CH,), jnp.float32),
            pltpu.VMEM((CH,), jnp.float32),
            pltpu.VMEM((CH,), jnp.int32),
            pltpu.VMEM((CH,), jnp.int32),
            pltpu.VMEM((CH,), jnp.int32),
            pltpu.VMEM((CH,), jnp.int32),
            pltpu.VMEM((CH, D), jnp.float32),
            pltpu.VMEM((CH, D), jnp.float32),
            pltpu.VMEM((HN,), jnp.float32),
            pltpu.VMEM((HN,), jnp.int32),
            pltpu.VMEM_SHARED((HN, D), jnp.float32),
            pltpu.SemaphoreType.DMA,
            pltpu.SemaphoreType.DMA,
            pltpu.SemaphoreType.DMA,
            pltpu.SemaphoreType.DMA,
        ],
        compiler_params=pltpu.CompilerParams(needs_layout_passes=False),
    )
    return fn(h, src3, dst3, s1, s2)


def _fin_body(acc_ref, rs_ref, o_ref):
    hp = acc_ref[...]
    rs = jnp.sum(rs_ref[...], axis=-1, keepdims=True)
    hp = hp / (rs + 1e-16)
    out = jnp.where(hp > 0, hp, jnp.exp(jnp.minimum(hp, 0.0)) - 1.0)
    u = out / SCALE
    nrm = jnp.maximum(
        jnp.sqrt(jnp.sum(u * u, axis=-1, keepdims=True)), 1e-15)
    p = jnp.tanh(nrm) * u / nrm
    pn = jnp.maximum(
        jnp.sqrt(jnp.sum(p * p, axis=-1, keepdims=True)), 1e-15)
    maxnorm = 1.0 - 1e-5
    o_ref[...] = jnp.where(pn > maxnorm, p / pn * maxnorm, p)


def _fin(acc, rs):
    return pl.pallas_call(
        _fin_body,
        grid=(N // MM_B,),
        in_specs=[
            pl.BlockSpec((MM_B, D), lambda i: (i, 0)),
            pl.BlockSpec((MM_B, NS), lambda i: (i, 0)),
        ],
        out_specs=pl.BlockSpec((MM_B, D), lambda i: (i, 0)),
        out_shape=jax.ShapeDtypeStruct((N, D), jnp.float32),
    )(acc, rs)


def kernel(input, edge_index, W, a):
    x = input.astype(jnp.float32)
    wt = W.T
    a12 = jnp.pad(a.reshape(2, D).T, ((0, 0), (0, 6)))
    h, sprj = _mm(x, wt, a12)
    s1 = sprj[:, 0]
    s2 = sprj[:, 1]
    src3 = edge_index[0].reshape(NS, NSC, SB, CH)
    dst3 = edge_index[1].reshape(NS, NSC, SB, CH)
    acc, rs = _sc_edge(h, src3, dst3, s1, s2)
    return _fin(acc, rs.reshape(NS, NP).T)


# speedup vs baseline: 9.8477x; 1.0443x over previous
import jax
import jax.numpy as jnp
from jax import lax
from jax.experimental import pallas as pl
from jax.experimental.pallas import tpu as pltpu
from jax.experimental.pallas import tpu_sc as plsc

N = 10000
E = 320000
D = 128
ALPHA = 0.2
SCALE = 10.0

NC = 2
NS = 16
L = 16
EPT = E // NS
CH = 80
NCHUNK = EPT // CH
SB = 50
NSC = NCHUNK // SB
NP = 10240
HN = NP // NC
RPT = HN // NS
MM_B = 1000


def _mm_body(x_ref, wt_ref, a12_ref, h_ref, s_ref):
    h = jnp.dot(x_ref[...], wt_ref[...], preferred_element_type=jnp.float32)
    h_ref[...] = h
    s_ref[...] = jnp.dot(h, a12_ref[...], preferred_element_type=jnp.float32)


def _mm(x, wt, a12):
    return pl.pallas_call(
        _mm_body,
        grid=(N // MM_B,),
        in_specs=[
            pl.BlockSpec((MM_B, D), lambda i: (i, 0)),
            pl.BlockSpec((D, D), lambda i: (0, 0)),
            pl.BlockSpec((D, 8), lambda i: (0, 0)),
        ],
        out_specs=[
            pl.BlockSpec((MM_B, D), lambda i: (i, 0)),
            pl.BlockSpec((MM_B, 8), lambda i: (i, 0)),
        ],
        out_shape=[
            jax.ShapeDtypeStruct((N, D), jnp.float32),
            jax.ShapeDtypeStruct((N, 8), jnp.float32),
        ],
    )(x, wt, a12)


def _sc_edge_body(h_hbm, src_hbm, dst_hbm, s1_hbm, s2_hbm,
                  out_hbm, rs_hbm,
                  s1_v, s2_v, srcv, dstv,
                  w_a, w_b, srcm_a, srcm_b, dstm_a, dstm_b,
                  rows_a, rows_b, rs_local, probe_v, cnt_v,
                  acc_sh, sg_a, sg_b, ss_a, ss_b):
    c = lax.axis_index("c")
    s = lax.axis_index("s")
    lane = lax.iota(jnp.int32, L)
    zero16 = jnp.zeros((L,), jnp.float32)
    lo = c * HN

    pltpu.sync_copy(s1_hbm, s1_v)
    pltpu.sync_copy(s2_hbm, s2_v)

    def _z16(i, carry):
        off = pl.multiple_of(i * L, L)
        rs_local[pl.ds(off, L)] = zero16
        return carry

    lax.fori_loop(0, HN // L, _z16, 0)

    def _zrow(i, carry):
        for u in range(D // L):
            rows_a[i, pl.ds(u * L, L)] = zero16
        return carry

    lax.fori_loop(0, CH, _zrow, 0)
    for k in range(RPT // CH):
        pltpu.sync_copy(rows_a, acc_sh.at[pl.ds(s * RPT + k * CH, CH)])
    plsc.subcore_barrier()

    neg1 = jnp.full((L,), -1, jnp.int32)

    def _masks_w(j, w_v, src_m, dst_m, cnt_ref, slot):
        for g in range(CH // L):
            src_m[pl.ds(g * L, L)] = neg1
            dst_m[pl.ds(g * L, L)] = neg1
        base = jnp.zeros((L,), jnp.int32)
        for g in range(CH // L):
            sl = pl.ds(g * L, L)
            sv = srcv[j, sl]
            dv = dstv[j, sl]
            inr = (sv >= lo) & (sv < lo + HN)
            shalf = sv - lo
            lg = plsc.load_gather(s1_v, [sv]) + plsc.load_gather(s2_v, [dv])
            w = jnp.exp(-jnp.where(lg > 0, lg, ALPHA * lg))
            pos = base + plsc.cumsum(inr.astype(jnp.int32)) - 1
            plsc.store_scatter(src_m, [pos], shalf, mask=inr)
            plsc.store_scatter(dst_m, [pos], dv, mask=inr)
            plsc.store_scatter(w_v, [pos], w, mask=inr)
            base = base + plsc.all_reduce_population_count(inr)
            si = jnp.where(inr, shalf, 0)

            def _rs_round(m):
                plsc.store_scatter(probe_v, [si], lane, mask=m)
                got = plsc.load_gather(probe_v, [si])
                win = m & (got == lane)
                plsc.addupdate_scatter(rs_local, [si], w, mask=win)
                return m & jnp.logical_not(win)

            lax.while_loop(jnp.any, _rs_round, _rs_round(inr))
        cnt_ref[slot] = jnp.max((base + 3) & ~3)

    def _scale(w_v, rows_v, cnt_ref, slot):
        @plsc.parallel_loop(0, cnt_ref[slot], unroll=4)
        def _row(r):
            wspl = plsc.load_gather(w_v, [jnp.full((L,), r, jnp.int32)])
            for u in range(D // L):
                rows_v[r, pl.ds(u * L, L)] = rows_v[r, pl.ds(u * L, L)] * wspl

    def _g_start(dst_m, rows_v, sem):
        pltpu.async_copy(
            h_hbm.at[plsc.Indices(dst_m, ignored_value=-1)], rows_v, sem)

    def _g_wait(dst_m, rows_v, sem):
        pltpu.make_async_copy(
            h_hbm.at[plsc.Indices(dst_m, ignored_value=-1)], rows_v, sem
        ).wait()

    def _s_start(src_m, rows_v, sem):
        pltpu.async_copy(
            rows_v, acc_sh.at[plsc.Indices(src_m, ignored_value=-1)], sem,
            add=True)

    def _s_wait(src_m, rows_v, sem):
        pltpu.make_async_copy(
            rows_v, acc_sh.at[plsc.Indices(src_m, ignored_value=-1)], sem
        ).wait()

    NPAIR = SB // 2
    for k in range(NSC):
        pltpu.sync_copy(src_hbm.at[s, k], srcv)
        pltpu.sync_copy(dst_hbm.at[s, k], dstv)

        _masks_w(0, w_a, srcm_a, dstm_a, cnt_v, 0)
        _g_start(dstm_a, rows_a, sg_a)
        _masks_w(1, w_b, srcm_b, dstm_b, cnt_v, 1)
        _g_start(dstm_b, rows_b, sg_b)

        def _pair(i, carry):
            _g_wait(dstm_a, rows_a, sg_a)
            _scale(w_a, rows_a, cnt_v, 0)
            _s_start(srcm_a, rows_a, ss_a)
            _g_wait(dstm_b, rows_b, sg_b)
            _scale(w_b, rows_b, cnt_v, 1)
            _s_start(srcm_b, rows_b, ss_b)

            @pl.when(i < NPAIR - 1)
            def _prep_next():
                _s_wait(srcm_a, rows_a, ss_a)
                _masks_w(2 * i + 2, w_a, srcm_a, dstm_a, cnt_v, 0)
                _g_start(dstm_a, rows_a, sg_a)
                _s_wait(srcm_b, rows_b, ss_b)
                _masks_w(2 * i + 3, w_b, srcm_b, dstm_b, cnt_v, 1)
                _g_start(dstm_b, rows_b, sg_b)

            return carry

        lax.fori_loop(0, NPAIR, _pair, 0)
        _s_wait(srcm_a, rows_a, ss_a)
        _s_wait(srcm_b, rows_b, ss_b)

    pltpu.sync_copy(rs_local, rs_hbm.at[s, 0, pl.ds(c * HN, HN)])
    plsc.subcore_barrier()
    pltpu.sync_copy(acc_sh.at[pl.ds(s * RPT, RPT)],
                    out_hbm.at[pl.ds(c * HN + s * RPT, RPT)])


def _sc_edge(h, src3, dst3, s1, s2):
    mesh = plsc.VectorSubcoreMesh(core_axis_name="c", subcore_axis_name="s",
                                  num_cores=NC, num_subcores=NS)
    fn = pl.kernel(
        _sc_edge_body,
        out_type=[
            jax.ShapeDtypeStruct((NP, D), jnp.float32),
            jax.ShapeDtypeStruct((NS, 1, NP), jnp.float32),
        ],
        mesh=mesh,
        scratch_types=[
            pltpu.VMEM((N,), jnp.float32),
            pltpu.VMEM((N,), jnp.float32),
            pltpu.VMEM((SB, CH), jnp.int32),
            pltpu.VMEM((SB, CH), jnp.int32),
            pltpu.VMEM((CH,), jnp.float32),
            pltpu.VMEM((CH,), jnp.float32),
            pltpu.VMEM((CH,), jnp.int32),
            pltpu.VMEM((CH,), jnp.int32),
            pltpu.VMEM((CH,), jnp.int32),
            pltpu.VMEM((CH,), jnp.int32),
            pltpu.VMEM((CH, D), jnp.float32),
            pltpu.VMEM((CH, D), jnp.float32),
            pltpu.VMEM((HN,), jnp.float32),
            pltpu.VMEM((HN,), jnp.int32),
            pltpu.SMEM((2,), jnp.int32),
            pltpu.VMEM_SHARED((HN, D), jnp.float32),
            pltpu.SemaphoreType.DMA,
            pltpu.SemaphoreType.DMA,
            pltpu.SemaphoreType.DMA,
            pltpu.SemaphoreType.DMA,
        ],
        compiler_params=pltpu.CompilerParams(needs_layout_passes=False),
    )
    return fn(h, src3, dst3, s1, s2)


def _fin_body(acc_ref, rs_ref, o_ref):
    hp = acc_ref[...]
    rs = jnp.sum(rs_ref[...], axis=-1, keepdims=True)
    hp = hp / (rs + 1e-16)
    out = jnp.where(hp > 0, hp, jnp.exp(jnp.minimum(hp, 0.0)) - 1.0)
    u = out / SCALE
    nrm = jnp.maximum(
        jnp.sqrt(jnp.sum(u * u, axis=-1, keepdims=True)), 1e-15)
    p = jnp.tanh(nrm) * u / nrm
    pn = jnp.maximum(
        jnp.sqrt(jnp.sum(p * p, axis=-1, keepdims=True)), 1e-15)
    maxnorm = 1.0 - 1e-5
    o_ref[...] = jnp.where(pn > maxnorm, p / pn * maxnorm, p)


def _fin(acc, rs):
    return pl.pallas_call(
        _fin_body,
        grid=(N // MM_B,),
        in_specs=[
            pl.BlockSpec((MM_B, D), lambda i: (i, 0)),
            pl.BlockSpec((MM_B, NS), lambda i: (i, 0)),
        ],
        out_specs=pl.BlockSpec((MM_B, D), lambda i: (i, 0)),
        out_shape=jax.ShapeDtypeStruct((N, D), jnp.float32),
    )(acc, rs)


def kernel(input, edge_index, W, a):
    x = input.astype(jnp.float32)
    wt = W.T
    a12 = jnp.pad(a.reshape(2, D).T, ((0, 0), (0, 6)))
    h, sprj = _mm(x, wt, a12)
    s1 = sprj[:, 0]
    s2 = sprj[:, 1]
    src3 = edge_index[0].reshape(NS, NSC, SB, CH)
    dst3 = edge_index[1].reshape(NS, NSC, SB, CH)
    acc, rs = _sc_edge(h, src3, dst3, s1, s2)
    return _fin(acc, rs.reshape(NS, NP).T)
